# Initial kernel scaffold; baseline (speedup 1.0000x reference)
import jax
import jax.numpy as jnp
from jax.experimental import pallas as pl


def kernel(h0, coord0, g0, edge_index, batch, params):
    raise NotImplementedError("write your pallas kernel here")


# speedup vs baseline: 1.2212x; 1.2212x over previous
import functools

import jax
import jax.numpy as jnp
from jax.experimental import pallas as pl
from jax.experimental.pallas import tpu as pltpu

HID = 16


def _silu(x):
    return x * jax.nn.sigmoid(x)


def _elu01(x):
    return jnp.where(x > 0, x, 0.1 * (jnp.exp(x) - 1.0))


def _matmul_body(x_ref, w_ref, o_ref):
    o_ref[...] = jnp.dot(x_ref[...], w_ref[...],
                         preferred_element_type=jnp.float32)


def _node_matmul(x, w, blk=2000):
    n, k = x.shape
    m = w.shape[1]
    grid = n // blk
    return pl.pallas_call(
        _matmul_body,
        grid=(grid,),
        in_specs=[
            pl.BlockSpec((blk, k), lambda i: (i, 0)),
            pl.BlockSpec((k, m), lambda i: (0, 0)),
        ],
        out_specs=pl.BlockSpec((blk, m), lambda i: (i, 0)),
        out_shape=jax.ShapeDtypeStruct((n, m), jnp.float32),
    )(x, w)


def _edge_chain_body(u_ref, v_ref, w2_ref, wc1_ref, stk_ref, ef_ref, tr_ref):
    u = u_ref[...]
    v = v_ref[...]
    stk = stk_ref[...]
    wr = stk[0:1, :]
    b1 = stk[1:2, :]
    b2 = stk[2:3, :]
    bc1 = stk[3:4, :]
    wc2 = stk[4:5, :]
    cd = u[:, HID:HID + 3] - v[:, HID:HID + 3]
    radial = jnp.sum(cd * cd, axis=1, keepdims=True)
    m = _silu(u[:, :HID] + v[:, :HID] + radial * wr + b1)
    ef = _silu(jnp.dot(m, w2_ref[...], preferred_element_type=jnp.float32) + b2)
    t = _silu(jnp.dot(ef, wc1_ref[...], preferred_element_type=jnp.float32) + bc1)
    coef = jnp.sum(t * wc2, axis=1, keepdims=True)
    ef_ref[...] = ef
    tr_ref[...] = jnp.concatenate(
        [cd * coef, jnp.zeros_like(coef)], axis=1)


def _edge_chain(ug, vg, w2, wc1, stk, blk=2000):
    e = ug.shape[0]
    grid = e // blk
    return pl.pallas_call(
        _edge_chain_body,
        grid=(grid,),
        in_specs=[
            pl.BlockSpec((blk, HID + 4), lambda i: (i, 0)),
            pl.BlockSpec((blk, HID + 4), lambda i: (i, 0)),
            pl.BlockSpec((HID, HID), lambda i: (0, 0)),
            pl.BlockSpec((HID, HID), lambda i: (0, 0)),
            pl.BlockSpec((8, HID), lambda i: (0, 0)),
        ],
        out_specs=[
            pl.BlockSpec((blk, HID), lambda i: (i, 0)),
            pl.BlockSpec((blk, 4), lambda i: (i, 0)),
        ],
        out_shape=[
            jax.ShapeDtypeStruct((e, HID), jnp.float32),
            jax.ShapeDtypeStruct((e, 4), jnp.float32),
        ],
    )(ug, vg, w2, wc1, stk)


def _node_update_body(residual, hwh_ref, agg_ref, hres_ref, wagg_ref,
                      w2_ref, stk_ref, out_ref):
    stk = stk_ref[...]
    bn1 = stk[0:1, :]
    bn2 = stk[1:2, :]
    z = _silu(hwh_ref[...] +
              jnp.dot(agg_ref[...], wagg_ref[...],
                      preferred_element_type=jnp.float32) + bn1)
    hn = jnp.dot(z, w2_ref[...], preferred_element_type=jnp.float32) + bn2
    if residual:
        hn = hn + hres_ref[...]
    out_ref[...] = hn


def _node_update(hwh, agg, hres, wagg, w2, stk, residual, blk=2000):
    n = hwh.shape[0]
    grid = n // blk
    return pl.pallas_call(
        functools.partial(_node_update_body, residual),
        grid=(grid,),
        in_specs=[
            pl.BlockSpec((blk, HID), lambda i: (i, 0)),
            pl.BlockSpec((blk, HID), lambda i: (i, 0)),
            pl.BlockSpec((blk, HID), lambda i: (i, 0)),
            pl.BlockSpec((HID, HID), lambda i: (0, 0)),
            pl.BlockSpec((HID, HID), lambda i: (0, 0)),
            pl.BlockSpec((8, HID), lambda i: (0, 0)),
        ],
        out_specs=pl.BlockSpec((blk, HID), lambda i: (i, 0)),
        out_shape=jax.ShapeDtypeStruct((n, HID), jnp.float32),
    )(hwh, agg, hres, wagg, w2, stk)


def _gin_z_body(x_ref, agg_ref, w_ref, b_ref, z_ref):
    z_ref[...] = (jnp.dot(x_ref[...] + agg_ref[...], w_ref[...],
                          preferred_element_type=jnp.float32) + b_ref[0:1, :])


def _gin_z(x, agg, w, b8, blk=2000):
    n = x.shape[0]
    return pl.pallas_call(
        _gin_z_body,
        grid=(n // blk,),
        in_specs=[
            pl.BlockSpec((blk, HID), lambda i: (i, 0)),
            pl.BlockSpec((blk, HID), lambda i: (i, 0)),
            pl.BlockSpec((HID, HID), lambda i: (0, 0)),
            pl.BlockSpec((8, HID), lambda i: (0, 0)),
        ],
        out_specs=pl.BlockSpec((blk, HID), lambda i: (i, 0)),
        out_shape=jax.ShapeDtypeStruct((n, HID), jnp.float32),
    )(x, agg, w, b8)


def _gin_out_body(z_ref, w2_ref, stk_ref, out_ref):
    stk = stk_ref[...]
    scale = stk[0:1, :]
    shift = stk[1:2, :]
    b2 = stk[2:3, :]
    zn = jax.nn.relu(z_ref[...] * scale + shift)
    out_ref[...] = _elu01(
        jnp.dot(zn, w2_ref[...], preferred_element_type=jnp.float32) + b2)


def _gin_out(z, w2, stk, blk=2000):
    n = z.shape[0]
    return pl.pallas_call(
        _gin_out_body,
        grid=(n // blk,),
        in_specs=[
            pl.BlockSpec((blk, HID), lambda i: (i, 0)),
            pl.BlockSpec((HID, HID), lambda i: (0, 0)),
            pl.BlockSpec((8, HID), lambda i: (0, 0)),
        ],
        out_specs=pl.BlockSpec((blk, HID), lambda i: (i, 0)),
        out_shape=jax.ShapeDtypeStruct((n, HID), jnp.float32),
    )(z, w2, stk)


def _classifier_body(x1_ref, x2_ref, g_ref, c1a_ref, c1b_ref, c2_ref,
                     stk_ref, out_ref):
    stk = stk_ref[...]
    c1g = stk[0:1, :]
    cb1 = stk[1:2, :]
    b2 = stk[2:3, :10]
    xh = _elu01(
        jnp.dot(x1_ref[...], c1a_ref[...], preferred_element_type=jnp.float32)
        + jnp.dot(x2_ref[...], c1b_ref[...], preferred_element_type=jnp.float32)
        + g_ref[...] * c1g + cb1)
    logits = jnp.dot(xh, c2_ref[...], preferred_element_type=jnp.float32) + b2
    mx = jnp.max(logits, axis=1, keepdims=True)
    ex = jnp.exp(logits - mx)
    out_ref[...] = ex / jnp.sum(ex, axis=1, keepdims=True)


def _classifier(x1, x2, g, c1a, c1b, c2, stk):
    b = x1.shape[0]
    return pl.pallas_call(
        _classifier_body,
        grid=(1,),
        in_specs=[
            pl.BlockSpec((b, HID), lambda i: (0, 0)),
            pl.BlockSpec((b, HID), lambda i: (0, 0)),
            pl.BlockSpec((b, 1), lambda i: (0, 0)),
            pl.BlockSpec((HID, HID), lambda i: (0, 0)),
            pl.BlockSpec((HID, HID), lambda i: (0, 0)),
            pl.BlockSpec((HID, 10), lambda i: (0, 0)),
            pl.BlockSpec((8, HID), lambda i: (0, 0)),
        ],
        out_specs=pl.BlockSpec((b, 10), lambda i: (0, 0)),
        out_shape=jax.ShapeDtypeStruct((b, 10), jnp.float32),
    )(x1, x2, g, c1a, c1b, c2, stk)


def _stack_rows(rows):
    out = jnp.zeros((8, HID), jnp.float32)
    for i, r in enumerate(rows):
        out = out.at[i, :r.shape[0]].set(r)
    return out


def _egcl(p, h, row, col, coord, inv_deg, residual):
    inp = h.shape[1]
    w1 = p["ew1"]["W"]
    wa, wb, wr = w1[:inp], w1[inp:2 * inp], w1[2 * inp]
    wh = p["nw1"]["W"][:inp]
    proj = _node_matmul(h, jnp.concatenate([wa, wb, wh], axis=1))
    coordp = jnp.concatenate(
        [coord, jnp.zeros((coord.shape[0], 1), jnp.float32)], axis=1)
    u = jnp.concatenate([proj[:, :HID], coordp], axis=1)
    v = jnp.concatenate([proj[:, HID:2 * HID], coordp], axis=1)
    ug = u[row]
    vg = v[col]
    stk = _stack_rows([wr, p["ew1"]["b"], p["ew2"]["b"], p["cw1"]["b"],
                       p["cw2"][:, 0]])
    ef, tr = _edge_chain(ug, vg, p["ew2"]["W"], p["cw1"]["W"], stk)
    agg = jax.ops.segment_sum(ef, row, num_segments=h.shape[0])
    aggt = jax.ops.segment_sum(tr, row, num_segments=h.shape[0])
    coord_new = coord + aggt[:, :3] * inv_deg
    nstk = _stack_rows([p["nw1"]["b"], p["nw2"]["b"]])
    hres = h if residual else jnp.zeros((h.shape[0], HID), jnp.float32)
    hn = _node_update(proj[:, 2 * HID:3 * HID], agg, hres,
                      p["nw1"]["W"][inp:], p["nw2"]["W"], nstk, residual)
    return hn, coord_new


def _gin(p, x, row, col):
    agg = jax.ops.segment_sum(x[row], col, num_segments=x.shape[0])
    b8 = _stack_rows([p["l1"]["b"]])
    z = _gin_z(x, agg, p["l1"]["W"], b8)
    mu = jnp.mean(z, axis=0)
    var = jnp.var(z, axis=0)
    inv = 1.0 / jnp.sqrt(var + 1e-5)
    scale = p["gamma"] * inv
    shift = p["beta"] - mu * scale
    stk = _stack_rows([scale, shift, p["l2"]["b"]])
    return _gin_out(z, p["l2"]["W"], stk)


def kernel(h0, coord0, g0, edge_index, batch, params):
    n = h0.shape[0]
    b = g0.shape[0]
    row, col = edge_index[0], edge_index[1]
    deg = jax.ops.segment_sum(jnp.ones((row.shape[0],), jnp.float32), row,
                              num_segments=n)
    inv_deg = (1.0 / jnp.maximum(deg, 1.0))[:, None]

    h, coord = _egcl(params["egcl0"], h0, row, col, coord0, inv_deg, False)
    h, coord = _egcl(params["egcl1"], h, row, col, coord, inv_deg, True)
    h = _gin(params["gin0"], h, row, col)
    h = _gin(params["gin1"], h, row, col)

    cnt = jax.ops.segment_sum(jnp.ones((n,), jnp.float32), batch,
                              num_segments=b)
    x1 = jax.ops.segment_sum(h, batch, num_segments=b) / \
        jnp.maximum(cnt, 1.0)[:, None]
    x2 = jax.ops.segment_max(h, batch, num_segments=b)

    c1w = params["c1"]["W"]
    stk = _stack_rows([c1w[2 * HID], params["c1"]["b"], params["c2"]["b"]])
    return _classifier(x1, x2, g0.reshape(b, 1), c1w[:HID], c1w[HID:2 * HID],
                       params["c2"]["W"], stk)


# speedup vs baseline: 4.7950x; 3.9265x over previous
import functools

import jax
import jax.numpy as jnp
from jax import lax
from jax.experimental import pallas as pl
from jax.experimental.pallas import tpu as pltpu
from jax.experimental.pallas import tpu_sc as plsc

HID = 16
CHUNK = 128
NW = 32


def _silu(x):
    return x * jax.nn.sigmoid(x)


def _elu01(x):
    return jnp.where(x > 0, x, 0.1 * (jnp.exp(x) - 1.0))


def _cdiv(a, b):
    return -(-a // b)


def _sc_mesh():
    return plsc.VectorSubcoreMesh(core_axis_name="c", subcore_axis_name="s")


def _wid():
    return lax.axis_index("s") * 2 + lax.axis_index("c")


def _sc_gather_uv(t, row2d, col2d, nchunks, slots):
    n, w = t.shape
    epad = slots * NW * CHUNK

    def body(t_hbm, row_hbm, col_hbm, ug_hbm, vg_hbm,
             idxr, idxc, ubuf, vbuf, sem):
        base = _wid() * slots
        nj = jnp.clip(nchunks - base, 0, slots)
        pltpu.sync_copy(row_hbm.at[pl.ds(base, slots)], idxr)
        pltpu.sync_copy(col_hbm.at[pl.ds(base, slots)], idxc)

        def step(j, carry):
            chunk = base + j
            pltpu.async_copy(t_hbm.at[idxr.at[j]], ubuf, sem).wait()
            pltpu.sync_copy(ubuf, ug_hbm.at[pl.ds(chunk * CHUNK, CHUNK)])
            pltpu.async_copy(t_hbm.at[idxc.at[j]], vbuf, sem).wait()
            pltpu.sync_copy(vbuf, vg_hbm.at[pl.ds(chunk * CHUNK, CHUNK)])
            return carry

        lax.fori_loop(0, nj, step, 0)

    f = pl.kernel(
        body,
        out_type=[
            jax.ShapeDtypeStruct((epad, w), jnp.float32),
            jax.ShapeDtypeStruct((epad, w), jnp.float32),
        ],
        mesh=_sc_mesh(),
        scratch_types=[
            pltpu.VMEM((slots, CHUNK), jnp.int32),
            pltpu.VMEM((slots, CHUNK), jnp.int32),
            pltpu.VMEM((CHUNK, w), jnp.float32),
            pltpu.VMEM((CHUNK, w), jnp.float32),
            pltpu.SemaphoreType.DMA,
        ],
    )
    return f(t, row2d, col2d)


def _sc_scatter(eftr, row2d, z128, nchunks, slots, npad):
    rpt = npad // 16

    def body(v_hbm, row_hbm, z_hbm, p, idxr, vbuf, acc):
        cid = lax.axis_index("c")
        sid = lax.axis_index("s")
        base = _wid() * slots
        nj = jnp.clip(nchunks - base, 0, slots)
        r0 = sid * rpt
        pltpu.sync_copy(z_hbm.at[pl.ds(r0, rpt)], acc.at[pl.ds(r0, rpt)])
        pltpu.sync_copy(row_hbm.at[pl.ds(base, slots)], idxr)
        plsc.subcore_barrier()

        def step(j, carry):
            chunk = base + j
            pltpu.sync_copy(v_hbm.at[pl.ds(chunk * CHUNK, CHUNK)], vbuf)
            pltpu.sync_copy(vbuf, acc.at[idxr.at[j]], add=True)
            return carry

        lax.fori_loop(0, nj, step, 0)
        plsc.subcore_barrier()
        pltpu.sync_copy(acc.at[pl.ds(r0, rpt)], p.at[cid, pl.ds(r0, rpt)])

    f = pl.kernel(
        body,
        out_type=[
            jax.ShapeDtypeStruct((2, npad, 128), jnp.float32),
        ],
        mesh=_sc_mesh(),
        scratch_types=[
            pltpu.VMEM((slots, CHUNK), jnp.int32),
            pltpu.VMEM((CHUNK, 128), jnp.float32),
            pltpu.VMEM_SHARED((npad, 128), jnp.float32),
        ],
    )
    return f(eftr, row2d, z128)


def _sc_gin_agg(x, row2d, col2d, z128, nchunks, slots, npad):
    rpt = npad // 16

    def body(x_hbm, row_hbm, col_hbm, z_hbm, p,
             idxr, idxc, buf, acc, sem):
        cid = lax.axis_index("c")
        sid = lax.axis_index("s")
        base = _wid() * slots
        nj = jnp.clip(nchunks - base, 0, slots)
        r0 = sid * rpt
        pltpu.sync_copy(z_hbm.at[pl.ds(r0, rpt)], acc.at[pl.ds(r0, rpt)])
        pltpu.sync_copy(row_hbm.at[pl.ds(base, slots)], idxr)
        pltpu.sync_copy(col_hbm.at[pl.ds(base, slots)], idxc)
        plsc.subcore_barrier()

        def step(j, carry):
            pltpu.async_copy(x_hbm.at[idxr.at[j]], buf, sem).wait()
            pltpu.sync_copy(buf, acc.at[idxc.at[j]], add=True)
            return carry

        lax.fori_loop(0, nj, step, 0)
        plsc.subcore_barrier()
        pltpu.sync_copy(acc.at[pl.ds(r0, rpt)], p.at[cid, pl.ds(r0, rpt)])

    f = pl.kernel(
        body,
        out_type=[
            jax.ShapeDtypeStruct((2, npad, 128), jnp.float32),
        ],
        mesh=_sc_mesh(),
        scratch_types=[
            pltpu.VMEM((slots, CHUNK), jnp.int32),
            pltpu.VMEM((slots, CHUNK), jnp.int32),
            pltpu.VMEM((CHUNK, 128), jnp.float32),
            pltpu.VMEM_SHARED((npad, 128), jnp.float32),
            pltpu.SemaphoreType.DMA,
        ],
    )
    return f(x, row2d, col2d, z128)


def _proj_body(h_ref, c4_ref, w_ref, t_ref, hwh_ref):
    proj = jnp.dot(h_ref[...], w_ref[...], preferred_element_type=jnp.float32)
    c4 = c4_ref[...]
    pad = jnp.zeros((proj.shape[0], 92), jnp.float32)
    t_ref[...] = jnp.concatenate([proj[:, :2 * HID], c4, pad], axis=1)
    hwh_ref[...] = proj[:, 2 * HID:3 * HID]


def _proj_assemble(h, c4, wcat, blk=2000):
    n, inp = h.shape
    return pl.pallas_call(
        _proj_body,
        grid=(n // blk,),
        in_specs=[
            pl.BlockSpec((blk, inp), lambda i: (i, 0)),
            pl.BlockSpec((blk, 4), lambda i: (i, 0)),
            pl.BlockSpec((inp, 3 * HID), lambda i: (0, 0)),
        ],
        out_specs=[
            pl.BlockSpec((blk, 128), lambda i: (i, 0)),
            pl.BlockSpec((blk, HID), lambda i: (i, 0)),
        ],
        out_shape=[
            jax.ShapeDtypeStruct((n, 128), jnp.float32),
            jax.ShapeDtypeStruct((n, HID), jnp.float32),
        ],
    )(h, c4, wcat)


def _edge_chain_body(u_ref, v_ref, w2_ref, wc1_ref, stk_ref, o_ref):
    u = u_ref[...]
    v = v_ref[...]
    stk = stk_ref[...]
    wr = stk[0:1, :]
    b1 = stk[1:2, :]
    b2 = stk[2:3, :]
    bc1 = stk[3:4, :]
    wc2 = stk[4:5, :]
    cd = u[:, 2 * HID:2 * HID + 3] - v[:, 2 * HID:2 * HID + 3]
    radial = jnp.sum(cd * cd, axis=1, keepdims=True)
    m = _silu(u[:, :HID] + v[:, HID:2 * HID] + radial * wr + b1)
    ef = _silu(jnp.dot(m, w2_ref[...], preferred_element_type=jnp.float32) + b2)
    t = _silu(jnp.dot(ef, wc1_ref[...], preferred_element_type=jnp.float32) + bc1)
    coef = jnp.sum(t * wc2, axis=1, keepdims=True)
    one = jnp.ones_like(coef)
    zpad = jnp.zeros((coef.shape[0], 108), jnp.float32)
    o_ref[...] = jnp.concatenate([ef, cd * coef, one, zpad], axis=1)


def _edge_chain(ug, vg, w2, wc1, stk, e, blk=2000):
    return pl.pallas_call(
        _edge_chain_body,
        grid=(e // blk,),
        in_specs=[
            pl.BlockSpec((blk, 128), lambda i: (i, 0)),
            pl.BlockSpec((blk, 128), lambda i: (i, 0)),
            pl.BlockSpec((HID, HID), lambda i: (0, 0)),
            pl.BlockSpec((HID, HID), lambda i: (0, 0)),
            pl.BlockSpec((8, HID), lambda i: (0, 0)),
        ],
        out_specs=pl.BlockSpec((blk, 128), lambda i: (i, 0)),
        out_shape=jax.ShapeDtypeStruct((e, 128), jnp.float32),
    )(ug, vg, w2, wc1, stk)


def _combine_node_body(residual, p_ref, c4_ref,
                       hwh_ref, hres_ref, wagg_ref, w2_ref, stk_ref,
                       h_ref, c4o_ref):
    stk = stk_ref[...]
    bn1 = stk[0:1, :]
    bn2 = stk[1:2, :]
    tot = p_ref[0] + p_ref[1]
    agg = tot[:, :HID]
    tr = tot[:, HID:HID + 4]
    inv = 1.0 / jnp.maximum(tr[:, 3:4], 1.0)
    c4o = c4_ref[...] + tr * inv
    lane = lax.broadcasted_iota(jnp.int32, c4o.shape, 1)
    c4o_ref[...] = jnp.where(lane == 3, 0.0, c4o)
    z = _silu(hwh_ref[...] +
              jnp.dot(agg, wagg_ref[...], preferred_element_type=jnp.float32)
              + bn1)
    hn = jnp.dot(z, w2_ref[...], preferred_element_type=jnp.float32) + bn2
    if residual:
        hn = hn + hres_ref[...]
    h_ref[...] = hn


def _combine_node(p, c4, hwh, hres, wagg, w2, stk,
                  residual, n, blk=2000):
    return pl.pallas_call(
        functools.partial(_combine_node_body, residual),
        grid=(n // blk,),
        in_specs=[
            pl.BlockSpec((2, blk, 128), lambda i: (0, i, 0)),
            pl.BlockSpec((blk, 4), lambda i: (i, 0)),
            pl.BlockSpec((blk, HID), lambda i: (i, 0)),
            pl.BlockSpec((blk, HID), lambda i: (i, 0)),
            pl.BlockSpec((HID, HID), lambda i: (0, 0)),
            pl.BlockSpec((HID, HID), lambda i: (0, 0)),
            pl.BlockSpec((8, HID), lambda i: (0, 0)),
        ],
        out_specs=[
            pl.BlockSpec((blk, HID), lambda i: (i, 0)),
            pl.BlockSpec((blk, 4), lambda i: (i, 0)),
        ],
        out_shape=[
            jax.ShapeDtypeStruct((n, HID), jnp.float32),
            jax.ShapeDtypeStruct((n, 4), jnp.float32),
        ],
    )(p, c4, hwh, hres, wagg, w2, stk)


def _gin_z_body(x_ref, p_ref, w_ref, b_ref, z_ref):
    agg = (p_ref[0] + p_ref[1])[:, :HID]
    z_ref[...] = (jnp.dot(x_ref[...] + agg, w_ref[...],
                          preferred_element_type=jnp.float32) + b_ref[0:1, :])


def _gin_z(x, p, w, b8, n, blk=2000):
    return pl.pallas_call(
        _gin_z_body,
        grid=(n // blk,),
        in_specs=[
            pl.BlockSpec((blk, HID), lambda i: (i, 0)),
            pl.BlockSpec((2, blk, 128), lambda i: (0, i, 0)),
            pl.BlockSpec((HID, HID), lambda i: (0, 0)),
            pl.BlockSpec((8, HID), lambda i: (0, 0)),
        ],
        out_specs=pl.BlockSpec((blk, HID), lambda i: (i, 0)),
        out_shape=jax.ShapeDtypeStruct((n, HID), jnp.float32),
    )(x, p, w, b8)


def _gin_out_body(z_ref, w2_ref, stk_ref, out_ref):
    stk = stk_ref[...]
    scale = stk[0:1, :]
    shift = stk[1:2, :]
    b2 = stk[2:3, :]
    zn = jax.nn.relu(z_ref[...] * scale + shift)
    out_ref[...] = _elu01(
        jnp.dot(zn, w2_ref[...], preferred_element_type=jnp.float32) + b2)


def _gin_out(z, w2, stk, blk=2000):
    n = z.shape[0]
    return pl.pallas_call(
        _gin_out_body,
        grid=(n // blk,),
        in_specs=[
            pl.BlockSpec((blk, HID), lambda i: (i, 0)),
            pl.BlockSpec((HID, HID), lambda i: (0, 0)),
            pl.BlockSpec((8, HID), lambda i: (0, 0)),
        ],
        out_specs=pl.BlockSpec((blk, HID), lambda i: (i, 0)),
        out_shape=jax.ShapeDtypeStruct((n, HID), jnp.float32),
    )(z, w2, stk)


def _pool_body(nb, batch_ref, h_ref, ht_ref, sum_ref, max_ref):
    i = pl.program_id(0)

    @pl.when(i == 0)
    def _():
        sum_ref[...] = jnp.zeros_like(sum_ref)
        max_ref[...] = jnp.full_like(max_ref, -3.4e38)

    seg = batch_ref[...]
    h = h_ref[...]
    ht = ht_ref[...]
    blk = h.shape[0]
    mask = (seg == lax.broadcasted_iota(jnp.int32, (nb, blk), 0))
    hx = jnp.concatenate(
        [h, jnp.ones((blk, 1), jnp.float32),
         jnp.zeros((blk, 15), jnp.float32)], axis=1)
    sum_ref[...] += jnp.dot(mask.astype(jnp.float32), hx,
                            preferred_element_type=jnp.float32)
    for f in range(HID):
        hf = jnp.broadcast_to(ht[f:f + 1, :], (nb, blk))
        cand = jnp.max(jnp.where(mask, hf, -3.4e38), axis=1, keepdims=True)
        max_ref[:, f:f + 1] = jnp.maximum(max_ref[:, f:f + 1], cand)


def _pool(batch, h, ht, nb, blk=2048):
    n = h.shape[0]
    return pl.pallas_call(
        functools.partial(_pool_body, nb),
        grid=(n // blk,),
        in_specs=[
            pl.BlockSpec((1, blk), lambda i: (0, i)),
            pl.BlockSpec((blk, HID), lambda i: (i, 0)),
            pl.BlockSpec((HID, blk), lambda i: (0, i)),
        ],
        out_specs=[
            pl.BlockSpec((nb, 2 * HID), lambda i: (0, 0)),
            pl.BlockSpec((nb, HID), lambda i: (0, 0)),
        ],
        out_shape=[
            jax.ShapeDtypeStruct((nb, 2 * HID), jnp.float32),
            jax.ShapeDtypeStruct((nb, HID), jnp.float32),
        ],
    )(batch.reshape(1, n), h, ht)


def _classifier_body(sum_ref, max_ref, g_ref, c1a_ref, c1b_ref, c2_ref,
                     stk_ref, out_ref):
    stk = stk_ref[...]
    c1g = stk[0:1, :]
    cb1 = stk[1:2, :]
    b2 = stk[2:3, :10]
    s = sum_ref[...]
    x1 = s[:, :HID] / jnp.maximum(s[:, HID:HID + 1], 1.0)
    xh = _elu01(
        jnp.dot(x1, c1a_ref[...], preferred_element_type=jnp.float32)
        + jnp.dot(max_ref[...], c1b_ref[...],
                  preferred_element_type=jnp.float32)
        + g_ref[...] * c1g + cb1)
    logits = jnp.dot(xh, c2_ref[...], preferred_element_type=jnp.float32) + b2
    mx = jnp.max(logits, axis=1, keepdims=True)
    ex = jnp.exp(logits - mx)
    out_ref[...] = ex / jnp.sum(ex, axis=1, keepdims=True)


def _classifier(sums, maxs, g, c1a, c1b, c2, stk):
    b = sums.shape[0]
    return pl.pallas_call(
        _classifier_body,
        grid=(1,),
        in_specs=[
            pl.BlockSpec((b, 2 * HID), lambda i: (0, 0)),
            pl.BlockSpec((b, HID), lambda i: (0, 0)),
            pl.BlockSpec((b, 1), lambda i: (0, 0)),
            pl.BlockSpec((HID, HID), lambda i: (0, 0)),
            pl.BlockSpec((HID, HID), lambda i: (0, 0)),
            pl.BlockSpec((HID, 10), lambda i: (0, 0)),
            pl.BlockSpec((8, HID), lambda i: (0, 0)),
        ],
        out_specs=pl.BlockSpec((b, 10), lambda i: (0, 0)),
        out_shape=jax.ShapeDtypeStruct((b, 10), jnp.float32),
    )(sums, maxs, g, c1a, c1b, c2, stk)


def _stack_rows(rows):
    out = jnp.zeros((8, HID), jnp.float32)
    for i, r in enumerate(rows):
        out = out.at[i, :r.shape[0]].set(r)
    return out


def _egcl(p, h, c4, row2d, col2d, z128, nchunks, slots, npad, residual):
    n, inp = h.shape
    w1 = p["ew1"]["W"]
    wa, wb, wr = w1[:inp], w1[inp:2 * inp], w1[2 * inp]
    wh = p["nw1"]["W"][:inp]
    t, hwh = _proj_assemble(h, c4, jnp.concatenate([wa, wb, wh], axis=1))
    ug, vg = _sc_gather_uv(t, row2d, col2d, nchunks, slots)
    stk = _stack_rows([wr, p["ew1"]["b"], p["ew2"]["b"], p["cw1"]["b"],
                       p["cw2"][:, 0]])
    eftr = _edge_chain(ug, vg, p["ew2"]["W"], p["cw1"]["W"], stk,
                       nchunks * CHUNK)
    (pagg,) = _sc_scatter(eftr, row2d, z128, nchunks, slots, npad)
    nstk = _stack_rows([p["nw1"]["b"], p["nw2"]["b"]])
    hres = h if residual else hwh
    return _combine_node(pagg, c4, hwh, hres,
                         p["nw1"]["W"][inp:], p["nw2"]["W"], nstk,
                         residual, n)


def _gin(p, x, row2d, col2d, z128, nchunks, slots, npad):
    n = x.shape[0]
    x128 = jnp.pad(x, ((0, 0), (0, 128 - HID)))
    (pagg,) = _sc_gin_agg(x128, row2d, col2d, z128, nchunks, slots, npad)
    b8 = _stack_rows([p["l1"]["b"]])
    z = _gin_z(x, pagg, p["l1"]["W"], b8, n)
    mu = jnp.mean(z, axis=0)
    var = jnp.var(z, axis=0)
    inv = 1.0 / jnp.sqrt(var + 1e-5)
    scale = p["gamma"] * inv
    shift = p["beta"] - mu * scale
    stk = _stack_rows([scale, shift, p["l2"]["b"]])
    return _gin_out(z, p["l2"]["W"], stk)


def kernel(h0, coord0, g0, edge_index, batch, params):
    n = h0.shape[0]
    e = edge_index.shape[1]
    b = g0.shape[0]
    nchunks = e // CHUNK
    slots = _cdiv(_cdiv(nchunks, NW), 8) * 8
    npad = _cdiv(n, 128) * 128

    row2d = jnp.pad(edge_index[0].reshape(nchunks, CHUNK),
                    ((0, slots * NW - nchunks), (0, 0)))
    col2d = jnp.pad(edge_index[1].reshape(nchunks, CHUNK),
                    ((0, slots * NW - nchunks), (0, 0)))
    z128 = jnp.zeros((npad, 128), jnp.float32)
    c4 = jnp.concatenate([coord0, jnp.zeros((n, 1), jnp.float32)], axis=1)

    h, c4 = _egcl(params["egcl0"], h0, c4, row2d, col2d, z128,
                  nchunks, slots, npad, False)
    h, c4 = _egcl(params["egcl1"], h, c4, row2d, col2d, z128,
                  nchunks, slots, npad, True)
    h = _gin(params["gin0"], h, row2d, col2d, z128, nchunks, slots, npad)
    h = _gin(params["gin1"], h, row2d, col2d, z128, nchunks, slots, npad)

    n_pool = _cdiv(n, 2048) * 2048
    batch_p = jnp.pad(batch, (0, n_pool - n), constant_values=b)
    h_p = jnp.pad(h, ((0, n_pool - n), (0, 0)))
    sums, maxs = _pool(batch_p, h_p, h_p.T, b)
    c1w = params["c1"]["W"]
    stk = _stack_rows([c1w[2 * HID], params["c1"]["b"], params["c2"]["b"]])
    return _classifier(sums, maxs, g0.reshape(b, 1), c1w[:HID],
                       c1w[HID:2 * HID], params["c2"]["W"], stk)


# speedup vs baseline: 5.1768x; 1.0796x over previous
import functools

import jax
import jax.numpy as jnp
from jax import lax
from jax.experimental import pallas as pl
from jax.experimental.pallas import tpu as pltpu
from jax.experimental.pallas import tpu_sc as plsc

HID = 16
CHUNK = 128
NW = 32


def _silu(x):
    return x * jax.nn.sigmoid(x)


def _elu01(x):
    return jnp.where(x > 0, x, 0.1 * (jnp.exp(x) - 1.0))


def _cdiv(a, b):
    return -(-a // b)


def _sc_mesh():
    return plsc.VectorSubcoreMesh(core_axis_name="c", subcore_axis_name="s")


def _wid():
    return lax.axis_index("s") * 2 + lax.axis_index("c")


def _sc_gather_uv(t, row2d, col2d, nchunks, slots):
    n, w = t.shape
    epad = slots * NW * CHUNK

    def body(t_hbm, row_hbm, col_hbm, ug_hbm, vg_hbm,
             idxr, idxc, ubuf, vbuf, semu, semv):
        base = _wid() * slots
        nj = jnp.clip(nchunks - base, 0, slots)
        pltpu.sync_copy(row_hbm.at[pl.ds(base, slots)], idxr)
        pltpu.sync_copy(col_hbm.at[pl.ds(base, slots)], idxc)

        def step(j, carry):
            chunk = base + j
            hu = pltpu.async_copy(t_hbm.at[idxr.at[j]], ubuf, semu)
            hv = pltpu.async_copy(t_hbm.at[idxc.at[j]], vbuf, semv)
            hu.wait()
            pltpu.sync_copy(ubuf, ug_hbm.at[pl.ds(chunk * CHUNK, CHUNK)])
            hv.wait()
            pltpu.sync_copy(vbuf, vg_hbm.at[pl.ds(chunk * CHUNK, CHUNK)])
            return carry

        lax.fori_loop(0, nj, step, 0)

    f = pl.kernel(
        body,
        out_type=[
            jax.ShapeDtypeStruct((epad, w), jnp.float32),
            jax.ShapeDtypeStruct((epad, w), jnp.float32),
        ],
        mesh=_sc_mesh(),
        scratch_types=[
            pltpu.VMEM((slots, CHUNK), jnp.int32),
            pltpu.VMEM((slots, CHUNK), jnp.int32),
            pltpu.VMEM((CHUNK, w), jnp.float32),
            pltpu.VMEM((CHUNK, w), jnp.float32),
            pltpu.SemaphoreType.DMA,
            pltpu.SemaphoreType.DMA,
        ],
    )
    return f(t, row2d, col2d)


def _sc_scatter(eftr, row2d, z128, nchunks, slots, npad):
    rpt = npad // 16

    def body(v_hbm, row_hbm, z_hbm, p, idxr, vbuf, acc):
        cid = lax.axis_index("c")
        sid = lax.axis_index("s")
        base = _wid() * slots
        nj = jnp.clip(nchunks - base, 0, slots)
        r0 = sid * rpt
        pltpu.sync_copy(z_hbm.at[pl.ds(r0, rpt)], acc.at[pl.ds(r0, rpt)])
        pltpu.sync_copy(row_hbm.at[pl.ds(base, slots)], idxr)
        plsc.subcore_barrier()

        def step(j, carry):
            chunk = base + j
            pltpu.sync_copy(v_hbm.at[pl.ds(chunk * CHUNK, CHUNK)], vbuf)
            pltpu.sync_copy(vbuf, acc.at[idxr.at[j]], add=True)
            return carry

        lax.fori_loop(0, nj, step, 0)
        plsc.subcore_barrier()
        pltpu.sync_copy(acc.at[pl.ds(r0, rpt)], p.at[cid, pl.ds(r0, rpt)])

    f = pl.kernel(
        body,
        out_type=[
            jax.ShapeDtypeStruct((2, npad, 128), jnp.float32),
        ],
        mesh=_sc_mesh(),
        scratch_types=[
            pltpu.VMEM((slots, CHUNK), jnp.int32),
            pltpu.VMEM((CHUNK, 128), jnp.float32),
            pltpu.VMEM_SHARED((npad, 128), jnp.float32),
        ],
    )
    return f(eftr, row2d, z128)


def _sc_gin_agg(x, row2d, col2d, z128, nchunks, slots, npad):
    rpt = npad // 16

    def body(x_hbm, row_hbm, col_hbm, z_hbm, p,
             idxr, idxc, buf, acc, sem):
        cid = lax.axis_index("c")
        sid = lax.axis_index("s")
        base = _wid() * slots
        nj = jnp.clip(nchunks - base, 0, slots)
        r0 = sid * rpt
        pltpu.sync_copy(z_hbm.at[pl.ds(r0, rpt)], acc.at[pl.ds(r0, rpt)])
        pltpu.sync_copy(row_hbm.at[pl.ds(base, slots)], idxr)
        pltpu.sync_copy(col_hbm.at[pl.ds(base, slots)], idxc)
        plsc.subcore_barrier()

        def step(j, carry):
            pltpu.async_copy(x_hbm.at[idxr.at[j]], buf, sem).wait()
            pltpu.sync_copy(buf, acc.at[idxc.at[j]], add=True)
            return carry

        lax.fori_loop(0, nj, step, 0)
        plsc.subcore_barrier()
        pltpu.sync_copy(acc.at[pl.ds(r0, rpt)], p.at[cid, pl.ds(r0, rpt)])

    f = pl.kernel(
        body,
        out_type=[
            jax.ShapeDtypeStruct((2, npad, 128), jnp.float32),
        ],
        mesh=_sc_mesh(),
        scratch_types=[
            pltpu.VMEM((slots, CHUNK), jnp.int32),
            pltpu.VMEM((slots, CHUNK), jnp.int32),
            pltpu.VMEM((CHUNK, 128), jnp.float32),
            pltpu.VMEM_SHARED((npad, 128), jnp.float32),
            pltpu.SemaphoreType.DMA,
        ],
    )
    return f(x, row2d, col2d, z128)


def _proj_body(h_ref, c4_ref, w_ref, t_ref, hwh_ref):
    proj = jnp.dot(h_ref[...], w_ref[...], preferred_element_type=jnp.float32)
    c4 = c4_ref[...]
    pad = jnp.zeros((proj.shape[0], 92), jnp.float32)
    t_ref[...] = jnp.concatenate([proj[:, :2 * HID], c4, pad], axis=1)
    hwh_ref[...] = proj[:, 2 * HID:3 * HID]


def _proj_assemble(h, c4, wcat, blk=2000):
    n, inp = h.shape
    return pl.pallas_call(
        _proj_body,
        grid=(n // blk,),
        in_specs=[
            pl.BlockSpec((blk, inp), lambda i: (i, 0)),
            pl.BlockSpec((blk, 4), lambda i: (i, 0)),
            pl.BlockSpec((inp, 3 * HID), lambda i: (0, 0)),
        ],
        out_specs=[
            pl.BlockSpec((blk, 128), lambda i: (i, 0)),
            pl.BlockSpec((blk, HID), lambda i: (i, 0)),
        ],
        out_shape=[
            jax.ShapeDtypeStruct((n, 128), jnp.float32),
            jax.ShapeDtypeStruct((n, HID), jnp.float32),
        ],
    )(h, c4, wcat)


def _edge_chain_body(u_ref, v_ref, w2_ref, wc1_ref, stk_ref, o_ref):
    u = u_ref[...]
    v = v_ref[...]
    stk = stk_ref[...]
    wr = stk[0:1, :]
    b1 = stk[1:2, :]
    b2 = stk[2:3, :]
    bc1 = stk[3:4, :]
    wc2 = stk[4:5, :]
    cd = u[:, 2 * HID:2 * HID + 3] - v[:, 2 * HID:2 * HID + 3]
    radial = jnp.sum(cd * cd, axis=1, keepdims=True)
    m = _silu(u[:, :HID] + v[:, HID:2 * HID] + radial * wr + b1)
    ef = _silu(jnp.dot(m, w2_ref[...], preferred_element_type=jnp.float32) + b2)
    t = _silu(jnp.dot(ef, wc1_ref[...], preferred_element_type=jnp.float32) + bc1)
    coef = jnp.sum(t * wc2, axis=1, keepdims=True)
    one = jnp.ones_like(coef)
    zpad = jnp.zeros((coef.shape[0], 108), jnp.float32)
    o_ref[...] = jnp.concatenate([ef, cd * coef, one, zpad], axis=1)


def _edge_chain(ug, vg, w2, wc1, stk, e, blk=2000):
    return pl.pallas_call(
        _edge_chain_body,
        grid=(e // blk,),
        in_specs=[
            pl.BlockSpec((blk, 128), lambda i: (i, 0)),
            pl.BlockSpec((blk, 128), lambda i: (i, 0)),
            pl.BlockSpec((HID, HID), lambda i: (0, 0)),
            pl.BlockSpec((HID, HID), lambda i: (0, 0)),
            pl.BlockSpec((8, HID), lambda i: (0, 0)),
        ],
        out_specs=pl.BlockSpec((blk, 128), lambda i: (i, 0)),
        out_shape=jax.ShapeDtypeStruct((e, 128), jnp.float32),
    )(ug, vg, w2, wc1, stk)


def _combine_node_body(residual, p_ref, c4_ref,
                       hwh_ref, hres_ref, wagg_ref, w2_ref, stk_ref,
                       h_ref, c4o_ref):
    stk = stk_ref[...]
    bn1 = stk[0:1, :]
    bn2 = stk[1:2, :]
    tot = p_ref[0] + p_ref[1]
    agg = tot[:, :HID]
    tr = tot[:, HID:HID + 4]
    inv = 1.0 / jnp.maximum(tr[:, 3:4], 1.0)
    c4o = c4_ref[...] + tr * inv
    lane = lax.broadcasted_iota(jnp.int32, c4o.shape, 1)
    c4o_ref[...] = jnp.where(lane == 3, 0.0, c4o)
    z = _silu(hwh_ref[...] +
              jnp.dot(agg, wagg_ref[...], preferred_element_type=jnp.float32)
              + bn1)
    hn = jnp.dot(z, w2_ref[...], preferred_element_type=jnp.float32) + bn2
    if residual:
        hn = hn + hres_ref[...]
    h_ref[...] = hn


def _combine_node(p, c4, hwh, hres, wagg, w2, stk,
                  residual, n, blk=2000):
    return pl.pallas_call(
        functools.partial(_combine_node_body, residual),
        grid=(n // blk,),
        in_specs=[
            pl.BlockSpec((2, blk, 128), lambda i: (0, i, 0)),
            pl.BlockSpec((blk, 4), lambda i: (i, 0)),
            pl.BlockSpec((blk, HID), lambda i: (i, 0)),
            pl.BlockSpec((blk, HID), lambda i: (i, 0)),
            pl.BlockSpec((HID, HID), lambda i: (0, 0)),
            pl.BlockSpec((HID, HID), lambda i: (0, 0)),
            pl.BlockSpec((8, HID), lambda i: (0, 0)),
        ],
        out_specs=[
            pl.BlockSpec((blk, HID), lambda i: (i, 0)),
            pl.BlockSpec((blk, 4), lambda i: (i, 0)),
        ],
        out_shape=[
            jax.ShapeDtypeStruct((n, HID), jnp.float32),
            jax.ShapeDtypeStruct((n, 4), jnp.float32),
        ],
    )(p, c4, hwh, hres, wagg, w2, stk)


def _gin_z_body(x_ref, p_ref, w_ref, b_ref, z_ref):
    agg = (p_ref[0] + p_ref[1])[:, :HID]
    z_ref[...] = (jnp.dot(x_ref[...] + agg, w_ref[...],
                          preferred_element_type=jnp.float32) + b_ref[0:1, :])


def _gin_z(x, p, w, b8, n, blk=2000):
    return pl.pallas_call(
        _gin_z_body,
        grid=(n // blk,),
        in_specs=[
            pl.BlockSpec((blk, HID), lambda i: (i, 0)),
            pl.BlockSpec((2, blk, 128), lambda i: (0, i, 0)),
            pl.BlockSpec((HID, HID), lambda i: (0, 0)),
            pl.BlockSpec((8, HID), lambda i: (0, 0)),
        ],
        out_specs=pl.BlockSpec((blk, HID), lambda i: (i, 0)),
        out_shape=jax.ShapeDtypeStruct((n, HID), jnp.float32),
    )(x, p, w, b8)


def _gin_out_body(z_ref, w2_ref, stk_ref, out_ref):
    stk = stk_ref[...]
    scale = stk[0:1, :]
    shift = stk[1:2, :]
    b2 = stk[2:3, :]
    zn = jax.nn.relu(z_ref[...] * scale + shift)
    out_ref[...] = _elu01(
        jnp.dot(zn, w2_ref[...], preferred_element_type=jnp.float32) + b2)


def _gin_out(z, w2, stk, blk=2000):
    n = z.shape[0]
    return pl.pallas_call(
        _gin_out_body,
        grid=(n // blk,),
        in_specs=[
            pl.BlockSpec((blk, HID), lambda i: (i, 0)),
            pl.BlockSpec((HID, HID), lambda i: (0, 0)),
            pl.BlockSpec((8, HID), lambda i: (0, 0)),
        ],
        out_specs=pl.BlockSpec((blk, HID), lambda i: (i, 0)),
        out_shape=jax.ShapeDtypeStruct((n, HID), jnp.float32),
    )(z, w2, stk)


def _pool_body(nb, batch_ref, h_ref, ht_ref, sum_ref, max_ref):
    i = pl.program_id(0)

    @pl.when(i == 0)
    def _():
        sum_ref[...] = jnp.zeros_like(sum_ref)
        max_ref[...] = jnp.full_like(max_ref, -3.4e38)

    seg = batch_ref[...]
    h = h_ref[...]
    ht = ht_ref[...]
    blk = h.shape[0]
    mask = (seg == lax.broadcasted_iota(jnp.int32, (nb, blk), 0))
    hx = jnp.concatenate(
        [h, jnp.ones((blk, 1), jnp.float32),
         jnp.zeros((blk, 15), jnp.float32)], axis=1)
    sum_ref[...] += jnp.dot(mask.astype(jnp.float32), hx,
                            preferred_element_type=jnp.float32)
    for f in range(HID):
        hf = jnp.broadcast_to(ht[f:f + 1, :], (nb, blk))
        cand = jnp.max(jnp.where(mask, hf, -3.4e38), axis=1, keepdims=True)
        max_ref[:, f:f + 1] = jnp.maximum(max_ref[:, f:f + 1], cand)


def _pool(batch, h, ht, nb, blk=2048):
    n = h.shape[0]
    return pl.pallas_call(
        functools.partial(_pool_body, nb),
        grid=(n // blk,),
        in_specs=[
            pl.BlockSpec((1, blk), lambda i: (0, i)),
            pl.BlockSpec((blk, HID), lambda i: (i, 0)),
            pl.BlockSpec((HID, blk), lambda i: (0, i)),
        ],
        out_specs=[
            pl.BlockSpec((nb, 2 * HID), lambda i: (0, 0)),
            pl.BlockSpec((nb, HID), lambda i: (0, 0)),
        ],
        out_shape=[
            jax.ShapeDtypeStruct((nb, 2 * HID), jnp.float32),
            jax.ShapeDtypeStruct((nb, HID), jnp.float32),
        ],
    )(batch.reshape(1, n), h, ht)


def _classifier_body(sum_ref, max_ref, g_ref, c1a_ref, c1b_ref, c2_ref,
                     stk_ref, out_ref):
    stk = stk_ref[...]
    c1g = stk[0:1, :]
    cb1 = stk[1:2, :]
    b2 = stk[2:3, :10]
    s = sum_ref[...]
    x1 = s[:, :HID] / jnp.maximum(s[:, HID:HID + 1], 1.0)
    xh = _elu01(
        jnp.dot(x1, c1a_ref[...], preferred_element_type=jnp.float32)
        + jnp.dot(max_ref[...], c1b_ref[...],
                  preferred_element_type=jnp.float32)
        + g_ref[...] * c1g + cb1)
    logits = jnp.dot(xh, c2_ref[...], preferred_element_type=jnp.float32) + b2
    mx = jnp.max(logits, axis=1, keepdims=True)
    ex = jnp.exp(logits - mx)
    out_ref[...] = ex / jnp.sum(ex, axis=1, keepdims=True)


def _classifier(sums, maxs, g, c1a, c1b, c2, stk):
    b = sums.shape[0]
    return pl.pallas_call(
        _classifier_body,
        grid=(1,),
        in_specs=[
            pl.BlockSpec((b, 2 * HID), lambda i: (0, 0)),
            pl.BlockSpec((b, HID), lambda i: (0, 0)),
            pl.BlockSpec((b, 1), lambda i: (0, 0)),
            pl.BlockSpec((HID, HID), lambda i: (0, 0)),
            pl.BlockSpec((HID, HID), lambda i: (0, 0)),
            pl.BlockSpec((HID, 10), lambda i: (0, 0)),
            pl.BlockSpec((8, HID), lambda i: (0, 0)),
        ],
        out_specs=pl.BlockSpec((b, 10), lambda i: (0, 0)),
        out_shape=jax.ShapeDtypeStruct((b, 10), jnp.float32),
    )(sums, maxs, g, c1a, c1b, c2, stk)


def _stack_rows(rows):
    out = jnp.zeros((8, HID), jnp.float32)
    for i, r in enumerate(rows):
        out = out.at[i, :r.shape[0]].set(r)
    return out


def _egcl(p, h, c4, row2d, col2d, z128, nchunks, slots, npad, residual):
    n, inp = h.shape
    w1 = p["ew1"]["W"]
    wa, wb, wr = w1[:inp], w1[inp:2 * inp], w1[2 * inp]
    wh = p["nw1"]["W"][:inp]
    t, hwh = _proj_assemble(h, c4, jnp.concatenate([wa, wb, wh], axis=1))
    ug, vg = _sc_gather_uv(t, row2d, col2d, nchunks, slots)
    stk = _stack_rows([wr, p["ew1"]["b"], p["ew2"]["b"], p["cw1"]["b"],
                       p["cw2"][:, 0]])
    eftr = _edge_chain(ug, vg, p["ew2"]["W"], p["cw1"]["W"], stk,
                       nchunks * CHUNK)
    (pagg,) = _sc_scatter(eftr, row2d, z128, nchunks, slots, npad)
    nstk = _stack_rows([p["nw1"]["b"], p["nw2"]["b"]])
    hres = h if residual else hwh
    return _combine_node(pagg, c4, hwh, hres,
                         p["nw1"]["W"][inp:], p["nw2"]["W"], nstk,
                         residual, n)


def _gin(p, x, row2d, col2d, z128, nchunks, slots, npad):
    n = x.shape[0]
    x128 = jnp.pad(x, ((0, 0), (0, 128 - HID)))
    (pagg,) = _sc_gin_agg(x128, row2d, col2d, z128, nchunks, slots, npad)
    b8 = _stack_rows([p["l1"]["b"]])
    z = _gin_z(x, pagg, p["l1"]["W"], b8, n)
    mu = jnp.mean(z, axis=0)
    var = jnp.var(z, axis=0)
    inv = 1.0 / jnp.sqrt(var + 1e-5)
    scale = p["gamma"] * inv
    shift = p["beta"] - mu * scale
    stk = _stack_rows([scale, shift, p["l2"]["b"]])
    return _gin_out(z, p["l2"]["W"], stk)


def kernel(h0, coord0, g0, edge_index, batch, params):
    n = h0.shape[0]
    e = edge_index.shape[1]
    b = g0.shape[0]
    nchunks = e // CHUNK
    slots = _cdiv(_cdiv(nchunks, NW), 8) * 8
    npad = _cdiv(n, 128) * 128

    row2d = jnp.pad(edge_index[0].reshape(nchunks, CHUNK),
                    ((0, slots * NW - nchunks), (0, 0)))
    col2d = jnp.pad(edge_index[1].reshape(nchunks, CHUNK),
                    ((0, slots * NW - nchunks), (0, 0)))
    z128 = jnp.zeros((npad, 128), jnp.float32)
    c4 = jnp.concatenate([coord0, jnp.zeros((n, 1), jnp.float32)], axis=1)

    h, c4 = _egcl(params["egcl0"], h0, c4, row2d, col2d, z128,
                  nchunks, slots, npad, False)
    h, c4 = _egcl(params["egcl1"], h, c4, row2d, col2d, z128,
                  nchunks, slots, npad, True)
    h = _gin(params["gin0"], h, row2d, col2d, z128, nchunks, slots, npad)
    h = _gin(params["gin1"], h, row2d, col2d, z128, nchunks, slots, npad)

    n_pool = _cdiv(n, 2048) * 2048
    batch_p = jnp.pad(batch, (0, n_pool - n), constant_values=b)
    h_p = jnp.pad(h, ((0, n_pool - n), (0, 0)))
    sums, maxs = _pool(batch_p, h_p, h_p.T, b)
    c1w = params["c1"]["W"]
    stk = _stack_rows([c1w[2 * HID], params["c1"]["b"], params["c2"]["b"]])
    return _classifier(sums, maxs, g0.reshape(b, 1), c1w[:HID],
                       c1w[HID:2 * HID], params["c2"]["W"], stk)
